# Initial kernel scaffold; baseline (speedup 1.0000x reference)
import jax
import jax.numpy as jnp
from jax.experimental import pallas as pl


def kernel(x, edge_index, Wa1, W11, b11, Wa2, W12, b12):
    raise NotImplementedError("write your pallas kernel here")


# speedup vs baseline: 20.5439x; 20.5439x over previous
import functools

import jax
import jax.numpy as jnp
from jax import lax
from jax.experimental import pallas as pl
from jax.experimental.pallas import tpu as pltpu
from jax.experimental.pallas import tpu_sc as plsc

N = 10000
E = 320000
EE = 2 * E
D = 128
NP = 10240
NC, NS, L = 2, 16, 16
W = NC * NS
BATCH = 128
EPT = 20224
NB = EPT // BATCH
HALF = EPT // 2
NBH = NB // 2
EP = W * EPT
PAD_SRC = N + 16
LSP = 16
TBL = LSP * N * (N + 32)
ROWS_PT = NP // W
XCH = ROWS_PT // 2
SLICE16 = NP // NS
HSL = SLICE16 // 2
NPW = NP // W

_mesh = plsc.VectorSubcoreMesh(core_axis_name="c", subcore_axis_name="s")
_cparams = pltpu.CompilerParams(needs_layout_passes=False)
_i32 = jnp.int32


def _wid():
  return (lax.axis_index("s").astype(_i32) * NC
          + lax.axis_index("c").astype(_i32))


def _bcast(ref, i):
  return ref[i, :]


def _zero_acc(acc_v):
  zero = jnp.zeros((L,), jnp.float32)

  @pl.loop(_i32(0), _i32(NP // (8 * L)))
  def _z(j):
    for c in range(8):
      acc_v[pl.ds(j * 8 * L + c * L, L)] = zero


def _scatter_add_safe(acc_v, idx, val):
  for lane in range(L):
    m = lax.iota(_i32, L) == lane
    plsc.addupdate_scatter(acc_v, [idx], val, mask=m)


def _keys_half(src_v, dst_v, h_off, key_v):
  @pl.loop(_i32(0), _i32(NBH))
  def _row(j):
    for c in range(BATCH // L):
      off = h_off + j * BATCH + c * L
      key_v[j, pl.ds(c * L, L)] = (
          src_v[pl.ds(off, L)] * (N * LSP) + dst_v[pl.ds(off, L)] * LSP)


@functools.partial(
    pl.kernel,
    out_type=(
        jax.ShapeDtypeStruct((NP,), jnp.float32),
        jax.ShapeDtypeStruct((NP,), jnp.float32),
        jax.ShapeDtypeStruct((TBL,), jnp.int32),
    ),
    mesh=_mesh,
    compiler_params=_cparams,
    scratch_types=(
        pltpu.VMEM((HALF,), jnp.int32),
        pltpu.VMEM((HALF,), jnp.int32),
        pltpu.VMEM((NBH, BATCH), jnp.int32),
        pltpu.VMEM((NBH, BATCH), jnp.int32),
        pltpu.VMEM((NBH, BATCH), jnp.int32),
        pltpu.VMEM((NBH, BATCH), jnp.int32),
        pltpu.VMEM((XCH * D,), jnp.float32),
        pltpu.VMEM((D, L), jnp.float32),
        pltpu.VMEM((ROWS_PT,), jnp.float32),
        pltpu.VMEM((ROWS_PT,), jnp.float32),
        pltpu.SemaphoreType.DMA,
        pltpu.SemaphoreType.DMA,
    ),
)
def _k2(src_hbm, dst_hbm, x_hbm, wb_hbm, y1_hbm, x0_hbm, tbl_hbm,
        src_v, dst_v, key0_v, eid0_v, key1_v, eid1_v, x_v, wb_v, y1_v, x0_v,
        esem, ssem):
  wid = _wid()
  ebase = wid * EPT
  iota = lax.iota(_i32, L)
  pltpu.sync_copy(wb_hbm, wb_v)

  for h, (key_v, eid_v) in enumerate(((key0_v, eid0_v), (key1_v, eid1_v))):
    hb = ebase + h * HALF
    pltpu.async_copy(src_hbm.at[pl.ds(hb, HALF)], src_v, esem)
    pltpu.async_copy(dst_hbm.at[pl.ds(hb, HALF)], dst_v, esem)
    pltpu.make_async_copy(src_hbm.at[pl.ds(hb, HALF)], src_v, esem).wait()
    pltpu.make_async_copy(dst_hbm.at[pl.ds(hb, HALF)], dst_v, esem).wait()

    _keys_half(src_v, dst_v, 0, key_v)

    @pl.loop(_i32(0), _i32(NBH))
    def _eids(j):
      for c in range(BATCH // L):
        eid_v[j, pl.ds(c * L, L)] = (
            jnp.full((L,), hb + j * BATCH + c * L, _i32) + iota)

    @pl.loop(_i32(0), _i32(NBH))
    def _fire(j):
      pltpu.async_copy(eid_v.at[j], tbl_hbm.at[key_v.at[j]], ssem)

  rbase = wid * ROWS_PT
  iota_d = iota * D
  for xh in range(2):
    pltpu.sync_copy(
        x_hbm.at[pl.ds((rbase + xh * XCH) * D, XCH * D)], x_v)

    @pl.loop(_i32(0), _i32(XCH // L))
    def _grp(g):
      base_idx = iota_d + g * (L * D)
      acc = jnp.zeros((L,), jnp.float32)
      x0 = jnp.zeros((L,), jnp.float32)
      for d in range(D):
        xv = plsc.load_gather(x_v, [base_idx + d])
        xt = lax.convert_element_type(
            lax.convert_element_type(xv, _i32), jnp.float32)
        if d == 0:
          x0 = xt
        else:
          acc = acc + xt * wb_v[d, :]
      y1_v[pl.ds(xh * XCH + g * L, L)] = acc
      x0_v[pl.ds(xh * XCH + g * L, L)] = x0

  pltpu.sync_copy(y1_v, y1_hbm.at[pl.ds(rbase, ROWS_PT)])
  pltpu.sync_copy(x0_v, x0_hbm.at[pl.ds(rbase, ROWS_PT)])

  for key_v, eid_v in ((key0_v, eid0_v), (key1_v, eid1_v)):

    @pl.loop(_i32(0), _i32(NBH))
    def _drain(j):
      pltpu.make_async_copy(eid_v.at[j], tbl_hbm.at[key_v.at[j]], ssem).wait()


@functools.partial(
    pl.kernel,
    out_type=(
        jax.ShapeDtypeStruct((W * NP,), jnp.float32),
        jax.ShapeDtypeStruct((EP,), jnp.float32),
    ),
    mesh=_mesh,
    compiler_params=_cparams,
    scratch_types=(
        pltpu.VMEM((EPT,), jnp.int32),
        pltpu.VMEM((EPT,), jnp.int32),
        pltpu.VMEM((NBH, BATCH), jnp.int32),
        pltpu.VMEM((NBH, BATCH), jnp.int32),
        pltpu.VMEM((NBH, BATCH), jnp.int32),
        pltpu.VMEM((NBH, BATCH), jnp.int32),
        pltpu.VMEM((HALF,), jnp.float32),
        pltpu.VMEM((N,), jnp.float32),
        pltpu.VMEM((NP,), jnp.float32),
        pltpu.SemaphoreType.DMA,
        pltpu.SemaphoreType.DMA,
    ),
)
def _k3(src_hbm, dst_hbm, tbl_hbm, y1_hbm, s1_hbm, mask_hbm,
        src_v, dst_v, key0_v, twin0_v, key1_v, twin1_v, maskf_v, y_v, acc_v,
        esem, gsem):
  wid = _wid()
  ebase = wid * EPT
  iota = lax.iota(_i32, L)
  pltpu.async_copy(src_hbm.at[pl.ds(ebase, EPT)], src_v, esem)
  pltpu.async_copy(dst_hbm.at[pl.ds(ebase, EPT)], dst_v, esem)
  pltpu.async_copy(y1_hbm.at[pl.ds(0, N)], y_v, esem)
  pltpu.make_async_copy(src_hbm.at[pl.ds(ebase, EPT)], src_v, esem).wait()
  pltpu.make_async_copy(dst_hbm.at[pl.ds(ebase, EPT)], dst_v, esem).wait()

  halves = ((key0_v, twin0_v, 0), (key1_v, twin1_v, HALF))
  for key_v, twin_v, h_off in halves:
    _keys_half(src_v, dst_v, h_off, key_v)

    @pl.loop(_i32(0), _i32(NBH))
    def _fire(j):
      pltpu.async_copy(tbl_hbm.at[key_v.at[j]], twin_v.at[j], gsem)

  _zero_acc(acc_v)
  pltpu.make_async_copy(y1_hbm.at[pl.ds(0, N)], y_v, esem).wait()

  for key_v, twin_v, h_off in halves:

    @pl.loop(_i32(0), _i32(NBH))
    def _drain(j):
      pltpu.make_async_copy(tbl_hbm.at[key_v.at[j]], twin_v.at[j], gsem).wait()

    @pl.loop(_i32(0), _i32(NBH))
    def _edge(j):
      for c in range(BATCH // L):
        off = j * BATCH + c * L
        s = src_v[pl.ds(h_off + off, L)]
        d = dst_v[pl.ds(h_off + off, L)]
        tw = twin_v[j, pl.ds(c * L, L)]
        eid = jnp.full((L,), ebase + h_off + off, _i32) + iota
        mf = jnp.where(tw == eid, jnp.float32(1.0), jnp.float32(0.0))
        maskf_v[pl.ds(off, L)] = mf
        yv = plsc.load_gather(y_v, [d])
        _scatter_add_safe(acc_v, s, yv * mf)

    pltpu.sync_copy(maskf_v, mask_hbm.at[pl.ds(ebase + h_off, HALF)])

  pltpu.sync_copy(acc_v, s1_hbm.at[pl.ds(wid * NP, NP)])


@functools.partial(
    pl.kernel,
    out_type=jax.ShapeDtypeStruct((W * NP,), jnp.float32),
    mesh=_mesh,
    compiler_params=_cparams,
    scratch_types=(
        pltpu.VMEM((EPT,), jnp.int32),
        pltpu.VMEM((EPT,), jnp.int32),
        pltpu.VMEM((EPT,), jnp.float32),
        pltpu.VMEM((NP,), jnp.float32),
        pltpu.VMEM((NP,), jnp.float32),
        pltpu.VMEM((8, L), jnp.float32),
        pltpu.VMEM((NP,), jnp.float32),
        pltpu.SemaphoreType.DMA,
    ),
)
def _k4(src_hbm, dst_hbm, mask_hbm, s1_hbm, par_hbm, s2_hbm,
        src_v, dst_v, maskf_v, ya_v, yb_v, par_v, acc_v, esem):
  wid = _wid()
  ebase = wid * EPT
  pltpu.async_copy(src_hbm.at[pl.ds(ebase, EPT)], src_v, esem)
  pltpu.async_copy(dst_hbm.at[pl.ds(ebase, EPT)], dst_v, esem)
  pltpu.async_copy(mask_hbm.at[pl.ds(ebase, EPT)], maskf_v, esem)
  pltpu.sync_copy(s1_hbm, ya_v)
  pltpu.sync_copy(par_hbm, par_v)
  b11 = _bcast(par_v, 0)
  w12 = _bcast(par_v, 1)

  @pl.loop(_i32(0), _i32(NP // (8 * L)))
  def _y2(j):
    for c in range(8):
      sl = pl.ds(j * 8 * L + c * L, L)
      yb_v[sl] = w12 * (ya_v[sl] + b11)

  _zero_acc(acc_v)
  pltpu.make_async_copy(src_hbm.at[pl.ds(ebase, EPT)], src_v, esem).wait()
  pltpu.make_async_copy(dst_hbm.at[pl.ds(ebase, EPT)], dst_v, esem).wait()
  pltpu.make_async_copy(mask_hbm.at[pl.ds(ebase, EPT)], maskf_v, esem).wait()

  @pl.loop(_i32(0), _i32(NB))
  def _edge(j):
    for c in range(BATCH // L):
      off = j * BATCH + c * L
      s = src_v[pl.ds(off, L)]
      d = dst_v[pl.ds(off, L)]
      mf = maskf_v[pl.ds(off, L)]
      yv = plsc.load_gather(yb_v, [d])
      _scatter_add_safe(acc_v, s, yv * mf)

  pltpu.sync_copy(acc_v, s2_hbm.at[pl.ds(wid * NP, NP)])


@functools.partial(
    pl.kernel,
    out_type=jax.ShapeDtypeStruct((NP,), jnp.float32),
    mesh=_mesh,
    compiler_params=_cparams,
    scratch_types=(
        pltpu.VMEM((W * NPW,), jnp.float32),
        pltpu.VMEM((NPW,), jnp.float32),
        pltpu.SemaphoreType.DMA,
    ),
)
def _kreduce(parts_hbm, out_hbm, rows_v, out_v, sem):
  wid = _wid()
  base = wid * NPW
  for slot in range(W):
    pltpu.async_copy(parts_hbm.at[pl.ds(_i32(slot * NP) + base, NPW)],
                     rows_v.at[pl.ds(slot * NPW, NPW)], sem)
  for slot in range(W):
    pltpu.make_async_copy(parts_hbm.at[pl.ds(_i32(slot * NP) + base, NPW)],
                          rows_v.at[pl.ds(slot * NPW, NPW)], sem).wait()
  zero = jnp.zeros((L,), jnp.float32)
  for v in range(NPW // L):
    out_v[pl.ds(v * L, L)] = zero
  for slot in range(W):
    for v in range(NPW // L):
      sl = pl.ds(v * L, L)
      out_v[sl] = out_v[sl] + rows_v[pl.ds(slot * NPW + v * L, L)]
  pltpu.sync_copy(out_v, out_hbm.at[pl.ds(base, NPW)])


@functools.partial(
    pl.kernel,
    out_type=jax.ShapeDtypeStruct((NP,), jnp.float32),
    mesh=_mesh,
    compiler_params=_cparams,
    scratch_types=(
        pltpu.VMEM((NPW,), jnp.float32),
        pltpu.VMEM((NPW,), jnp.float32),
        pltpu.VMEM((NPW,), jnp.float32),
        pltpu.VMEM((8, L), jnp.float32),
        pltpu.VMEM((NPW,), jnp.float32),
    ),
)
def _k5(x0_hbm, s1_hbm, s2_hbm, par_hbm, out_hbm,
        x0_v, s1_v, s2_v, par_v, o_v):
  wid = _wid()
  base = wid * NPW
  sl = pl.ds(base, NPW)
  pltpu.sync_copy(x0_hbm.at[sl], x0_v)
  pltpu.sync_copy(s1_hbm.at[sl], s1_v)
  pltpu.sync_copy(s2_hbm.at[sl], s2_v)
  pltpu.sync_copy(par_hbm, par_v)
  b11 = _bcast(par_v, 0)
  wa1 = _bcast(par_v, 2)
  wa2 = _bcast(par_v, 3)
  b12 = _bcast(par_v, 4)
  for v in range(NPW // L):
    s = pl.ds(v * L, L)
    t1 = s1_v[s] + b11
    r1 = wa1 * x0_v[s] + t1
    o_v[s] = wa2 * r1 + s2_v[s] + b12
  pltpu.sync_copy(o_v, out_hbm.at[sl])


def kernel(x, edge_index, Wa1, W11, b11, Wa2, W12, b12):
  x = x.astype(jnp.float32)
  ei = edge_index.astype(jnp.int32)
  npad = EP - EE
  src_all = jnp.concatenate(
      [ei[0], ei[1], N + (jnp.arange(npad, dtype=jnp.int32) % (NP - N))])
  dst_all = jnp.concatenate(
      [ei[1], ei[0], (jnp.arange(npad, dtype=jnp.int32) * 17) % N])
  xflat = jnp.concatenate(
      [x, jnp.zeros((NP - N, D), jnp.float32)], axis=0).reshape(-1)
  wrow = jnp.concatenate([jnp.zeros((1,), jnp.float32), W11[0]])
  wb = jnp.broadcast_to(wrow[:, None], (D, L)).astype(jnp.float32)
  pvec = jnp.stack([
      b11[0], W12[0, 0], Wa1[0, 0], Wa2[0, 0], b12[0],
      jnp.float32(0), jnp.float32(0), jnp.float32(0)]).astype(jnp.float32)
  params = jnp.broadcast_to(pvec[:, None], (8, L))

  y1, x0t, tbl = _k2(src_all, dst_all, xflat, wb)
  s1p, maskf = _k3(src_all, dst_all, tbl, y1)
  s1 = _kreduce(s1p)
  s2p = _k4(src_all, dst_all, maskf, s1, params)
  s2 = _kreduce(s2p)
  out = _k5(x0t, s1, s2, params)
  return out[:N]
